# baseline (speedup 1.0000x reference)
import jax
import jax.numpy as jnp
import numpy as np
from jax import lax
from jax.experimental import pallas as pl
from jax.experimental.pallas import tpu as pltpu
from jax.experimental.pallas import tpu_sc as plsc

EMBED_DIM = 32
SCALE = float(np.sqrt(np.float32(EMBED_DIM)))

NROWS = 16384
SEQ = 50
NUM_WORKERS = 32
B_PER_W = NROWS // NUM_WORKERS
JBLK = 128
J_PER_W = B_PER_W // JBLK
N_CHUNKS = SEQ * J_PER_W
NBUF = 4


def _embed_body(xt_hbm, table_hbm, out_hbm, idx_v, g_v, t_v, *sems):
    gsems = sems[:NBUF]
    osems = sems[NBUF:]
    wid = lax.axis_index("s") * 2 + lax.axis_index("c")
    b0 = wid * B_PER_W

    for k in range(J_PER_W):
        pltpu.sync_copy(xt_hbm.at[:, pl.ds(b0 + k * JBLK, JBLK)],
                        idx_v.at[pl.ds(k * SEQ, SEQ)])

    def fire_gather(c, b):
        pltpu.async_copy(table_hbm.at[idx_v.at[c]], g_v.at[b], gsems[b])

    def wait_gather(c, b):
        pltpu.make_async_copy(
            table_hbm.at[idx_v.at[c]], g_v.at[b], gsems[b]).wait()

    def out_slices(c):
        k = c // SEQ
        s = c - k * SEQ
        return s, (wid * J_PER_W + k)

    def fire_out(c, b):
        s, j = out_slices(c)
        pltpu.async_copy(t_v.at[b], out_hbm.at[s, :, j], osems[b])

    def wait_out(c, b):
        s, j = out_slices(c)
        pltpu.make_async_copy(
            t_v.at[b], out_hbm.at[s, :, j], osems[b]).wait()

    def transpose_scale(b):
        base_iota = lax.iota(jnp.int32, 16)
        for i in range(4):
            for r in range(8):
                d = 8 * i + r
                dcol = jnp.full((16,), d, jnp.int32)

                def lane_blk(m, carry):
                    rows = base_iota + m * 16
                    vals = plsc.load_gather(g_v.at[b], [rows, dcol])
                    t_v[b, i, r, pl.ds(m * 16, 16)] = vals * SCALE
                    return carry

                lax.fori_loop(0, 8, lane_blk, 0, unroll=8)

    for b in range(NBUF):
        fire_gather(b, b)

    @pl.loop(0, N_CHUNKS, step=NBUF)
    def step(c0):
        for b in range(NBUF):
            c = c0 + b
            wait_gather(c, b)

            @pl.when(c >= NBUF)
            def _():
                wait_out(c - NBUF, b)

            transpose_scale(b)
            fire_out(c, b)

            @pl.when(c + NBUF < N_CHUNKS)
            def _():
                fire_gather(c + NBUF, b)

    for b in range(NBUF):
        wait_out(N_CHUNKS - NBUF + b, b)


@jax.jit
def _embed(xt, table):
    mesh = plsc.VectorSubcoreMesh(core_axis_name="c", subcore_axis_name="s")
    f = pl.kernel(
        _embed_body,
        mesh=mesh,
        out_type=jax.ShapeDtypeStruct((SEQ, 4, NROWS // JBLK, 8, JBLK),
                                      jnp.float32),
        scratch_types=[
            pltpu.VMEM((N_CHUNKS, JBLK), jnp.int32),
            pltpu.VMEM((NBUF, JBLK, EMBED_DIM), jnp.float32),
            pltpu.VMEM((NBUF, 4, 8, JBLK), jnp.float32),
        ] + [pltpu.SemaphoreType.DMA] * (2 * NBUF),
        compiler_params=pltpu.CompilerParams(
            use_tc_tiling_on_sc=False, needs_layout_passes=False),
    )
    return f(xt, table)


def kernel(x, table):
    xt = x.T.astype(jnp.int32)
    x5 = _embed(xt, table)
    out = jnp.transpose(x5, (2, 4, 0, 1, 3)).reshape(NROWS, SEQ, EMBED_DIM)
    return out


# speedup vs baseline: 1.6720x; 1.6720x over previous
import jax
import jax.numpy as jnp
import numpy as np
from jax import lax
from jax.experimental import pallas as pl
from jax.experimental.pallas import tpu as pltpu
from jax.experimental.pallas import tpu_sc as plsc

EMBED_DIM = 32
SCALE = float(np.sqrt(np.float32(EMBED_DIM)))

NROWS = 16384
SEQ = 50
NUM_WORKERS = 32
B_PER_W = NROWS // NUM_WORKERS
JBLK = 128
J_PER_W = B_PER_W // JBLK
N_CHUNKS = SEQ * J_PER_W
NBUF = 4


def _embed_body(xt_hbm, table_hbm, out_hbm, idx_v, g_v, t_v, *sems):
    gsems = sems[:NBUF]
    osems = sems[NBUF:]
    wid = lax.axis_index("s") * 2 + lax.axis_index("c")
    b0 = wid * B_PER_W

    for k in range(J_PER_W):
        pltpu.sync_copy(xt_hbm.at[:, pl.ds(b0 + k * JBLK, JBLK)],
                        idx_v.at[pl.ds(k * SEQ, SEQ)])

    def fire_gather(c, b):
        pltpu.async_copy(table_hbm.at[idx_v.at[c]], g_v.at[b], gsems[b])

    def wait_gather(c, b):
        pltpu.make_async_copy(
            table_hbm.at[idx_v.at[c]], g_v.at[b], gsems[b]).wait()

    def out_slices(c):
        k = c // SEQ
        s = c - k * SEQ
        return s, (wid * J_PER_W + k)

    def fire_out(c, b):
        s, j = out_slices(c)
        pltpu.async_copy(
            t_v.at[b, :, :, pl.ds(0, JBLK)], out_hbm.at[s, :, j], osems[b])

    def wait_out(c, b):
        s, j = out_slices(c)
        pltpu.make_async_copy(
            t_v.at[b, :, :, pl.ds(0, JBLK)], out_hbm.at[s, :, j],
            osems[b]).wait()

    def transpose_scale(b):
        iota = lax.iota(jnp.int32, 16)
        didx = [iota, iota + 16]
        ci = [d >> 3 for d in didx]
        cr = [d & 7 for d in didx]

        def tok(l, carry):
            cl = jnp.full((16,), l, jnp.int32)
            for h in range(2):
                vals = g_v[b, l, pl.ds(16 * h, 16)]
                plsc.store_scatter(t_v.at[b], [ci[h], cr[h], cl],
                                   vals * SCALE)
            return carry

        lax.fori_loop(0, JBLK, tok, 0, unroll=8)

    for b in range(NBUF):
        fire_gather(b, b)

    @pl.loop(0, N_CHUNKS, step=NBUF)
    def step(c0):
        for b in range(NBUF):
            c = c0 + b
            wait_gather(c, b)

            @pl.when(c >= NBUF)
            def _():
                wait_out(c - NBUF, b)

            transpose_scale(b)
            fire_out(c, b)

            @pl.when(c + NBUF < N_CHUNKS)
            def _():
                fire_gather(c + NBUF, b)

    for b in range(NBUF):
        wait_out(N_CHUNKS - NBUF + b, b)


@jax.jit
def _embed(xt, table):
    mesh = plsc.VectorSubcoreMesh(core_axis_name="c", subcore_axis_name="s")
    f = pl.kernel(
        _embed_body,
        mesh=mesh,
        out_type=jax.ShapeDtypeStruct((SEQ, 4, NROWS // JBLK, 8, JBLK),
                                      jnp.float32),
        scratch_types=[
            pltpu.VMEM((N_CHUNKS, JBLK), jnp.int32),
            pltpu.VMEM((NBUF, JBLK, EMBED_DIM), jnp.float32),
            pltpu.VMEM((NBUF, 4, 8, JBLK + 1), jnp.float32),
        ] + [pltpu.SemaphoreType.DMA] * (2 * NBUF),
        compiler_params=pltpu.CompilerParams(
            use_tc_tiling_on_sc=False, needs_layout_passes=False),
    )
    return f(xt, table)


def kernel(x, table):
    xt = x.T.astype(jnp.int32)
    x5 = _embed(xt, table)
    out = jnp.transpose(x5, (2, 4, 0, 1, 3)).reshape(NROWS, SEQ, EMBED_DIM)
    return out
